# baseline (speedup 1.0000x reference)
import functools

import jax
import jax.numpy as jnp
from jax import lax
from jax.experimental import pallas as pl
from jax.experimental.pallas import tpu as pltpu
from jax.experimental.pallas import tpu_sc as plsc

VOCAB = 100000
EMBED = 128
HIDDEN = 256
TOKENS = 4096 * 50

_NW = 32
_TPW = TOKENS // _NW
_CHUNK = 128
_NCHUNK = _TPW // _CHUNK
_NBUF = 2


def _gather_body(table_hbm, idx_hbm, out_hbm, idx_v, bufs, gsems, ssems):
    wid = lax.axis_index("s") * 2 + lax.axis_index("c")
    base = wid * _TPW

    pltpu.sync_copy(idx_hbm.at[wid], idx_v)

    def gather_op(chunk, b):
        return pltpu.make_async_copy(
            table_hbm.at[idx_v.at[chunk]], bufs[b], gsems[b])

    def scatter_op(chunk, b):
        off = base + chunk * _CHUNK
        return pltpu.make_async_copy(
            bufs[b], out_hbm.at[pl.ds(off, _CHUNK)], ssems[b])

    for b in range(_NBUF):
        gather_op(b, b).start()

    def group(g, carry):
        for b in range(_NBUF):
            i = g * _NBUF + b
            gather_op(i, b).wait()
            scatter_op(i, b).start()
            scatter_op(i, b).wait()
            gather_op(i + _NBUF, b).start()
        return carry

    lax.fori_loop(0, (_NCHUNK - _NBUF) // _NBUF, group, 0)

    for b in range(_NBUF):
        i = _NCHUNK - _NBUF + b
        gather_op(i, b).wait()
        scatter_op(i, b).start()
    for b in range(_NBUF):
        i = _NCHUNK - _NBUF + b
        scatter_op(i, b).wait()


_gather = functools.partial(
    pl.kernel,
    out_type=jax.ShapeDtypeStruct((TOKENS, EMBED), jnp.float32),
    mesh=plsc.VectorSubcoreMesh(core_axis_name="c", subcore_axis_name="s"),
    scratch_types=[
        pltpu.VMEM((_NCHUNK, _CHUNK), jnp.int32),
        [pltpu.VMEM((_CHUNK, EMBED), jnp.float32) for _ in range(_NBUF)],
        [pltpu.SemaphoreType.DMA for _ in range(_NBUF)],
        [pltpu.SemaphoreType.DMA for _ in range(_NBUF)],
    ],
)(_gather_body)


_BATCH = 4096
_SEQ = 50
_TB = 3200


def _proj_body(e_ref, w_ref, b_ref, o_ref):
    acc = jnp.dot(e_ref[...], w_ref[...], preferred_element_type=jnp.float32)
    o_ref[...] = jnp.maximum(acc + b_ref[...], 0.0)


def _project(embs, W, b):
    return pl.pallas_call(
        _proj_body,
        grid=(TOKENS // _TB,),
        in_specs=[
            pl.BlockSpec((_TB, EMBED), lambda i: (i, 0)),
            pl.BlockSpec((EMBED, HIDDEN), lambda i: (0, 0)),
            pl.BlockSpec((1, HIDDEN), lambda i: (0, 0)),
        ],
        out_specs=pl.BlockSpec((_TB, HIDDEN), lambda i: (i, 0)),
        out_shape=jax.ShapeDtypeStruct((TOKENS, HIDDEN), jnp.float32),
    )(embs, W, b.reshape(1, HIDDEN))


def kernel(X, table, W, b):
    idx = X.reshape(_NW, _NCHUNK, _CHUNK).astype(jnp.int32)
    embs = _gather(table, idx)
    return _project(embs, W, b).reshape(_BATCH, _SEQ, HIDDEN)


# speedup vs baseline: 1.4867x; 1.4867x over previous
import functools

import jax
import jax.numpy as jnp
from jax import lax
from jax.experimental import pallas as pl
from jax.experimental.pallas import tpu as pltpu
from jax.experimental.pallas import tpu_sc as plsc

VOCAB = 100000
EMBED = 128
HIDDEN = 256
BATCH = 4096
SEQ = 50
PSEQ = 56
PTOK = BATCH * PSEQ

_NW = 32
_TPW = PTOK // _NW
_CHUNK = 112
_NCHUNK = _TPW // _CHUNK
_NBUF = 2


def _gather_body(table_hbm, idx_hbm, out_hbm, idx_v, bufs, gsems, ssems):
    wid = lax.axis_index("s") * 2 + lax.axis_index("c")
    base = wid * _TPW

    pltpu.sync_copy(idx_hbm.at[wid], idx_v)

    def gather_op(chunk, b):
        return pltpu.make_async_copy(
            table_hbm.at[idx_v.at[chunk]], bufs[b], gsems[b])

    def scatter_op(chunk, b):
        off = base + chunk * _CHUNK
        return pltpu.make_async_copy(
            bufs[b], out_hbm.at[pl.ds(off, _CHUNK)], ssems[b])

    for b in range(_NBUF):
        gather_op(b, b).start()

    def group(g, carry):
        for b in range(_NBUF):
            i = g * _NBUF + b
            gather_op(i, b).wait()
            scatter_op(i, b).start()
            scatter_op(i, b).wait()
            gather_op(i + _NBUF, b).start()
        return carry

    lax.fori_loop(0, (_NCHUNK - _NBUF) // _NBUF, group, 0)

    for b in range(_NBUF):
        i = _NCHUNK - _NBUF + b
        gather_op(i, b).wait()
        scatter_op(i, b).start()
    for b in range(_NBUF):
        i = _NCHUNK - _NBUF + b
        scatter_op(i, b).wait()


_gather = functools.partial(
    pl.kernel,
    out_type=jax.ShapeDtypeStruct((PTOK, EMBED), jnp.float32),
    mesh=plsc.VectorSubcoreMesh(core_axis_name="c", subcore_axis_name="s"),
    scratch_types=[
        pltpu.VMEM((_NCHUNK, _CHUNK), jnp.int32),
        [pltpu.VMEM((_CHUNK, EMBED), jnp.float32) for _ in range(_NBUF)],
        [pltpu.SemaphoreType.DMA for _ in range(_NBUF)],
        [pltpu.SemaphoreType.DMA for _ in range(_NBUF)],
    ],
)(_gather_body)


_BB = 64


def _proj_body(e_ref, w_ref, b_ref, o_ref):
    acc = jnp.dot(e_ref[...], w_ref[...], preferred_element_type=jnp.float32)
    act = jnp.maximum(acc + b_ref[...], 0.0)
    o_ref[...] = act.reshape(_BB, PSEQ, HIDDEN)[:, :SEQ, :]


def _project(embs, W, b):
    return pl.pallas_call(
        _proj_body,
        grid=(BATCH // _BB,),
        in_specs=[
            pl.BlockSpec((_BB * PSEQ, EMBED), lambda i: (i, 0)),
            pl.BlockSpec((EMBED, HIDDEN), lambda i: (0, 0)),
            pl.BlockSpec((1, HIDDEN), lambda i: (0, 0)),
        ],
        out_specs=pl.BlockSpec((_BB, SEQ, HIDDEN), lambda i: (i, 0, 0)),
        out_shape=jax.ShapeDtypeStruct((BATCH, SEQ, HIDDEN), jnp.float32),
    )(embs, W, b.reshape(1, HIDDEN))


def kernel(X, table, W, b):
    Xp = jnp.pad(X, ((0, 0), (0, PSEQ - SEQ)), mode="edge")
    idx = Xp.reshape(_NW, _NCHUNK, _CHUNK).astype(jnp.int32)
    embs = _gather(table, idx)
    return _project(embs, W, b)


# speedup vs baseline: 2.7549x; 1.8530x over previous
import functools

import jax
import jax.numpy as jnp
from jax import lax
from jax.experimental import pallas as pl
from jax.experimental.pallas import tpu as pltpu
from jax.experimental.pallas import tpu_sc as plsc

VOCAB = 100000
EMBED = 128
HIDDEN = 256
BATCH = 4096
SEQ = 50
TOKENS = BATCH * SEQ

_NW = 32
_TPW = TOKENS // _NW
_CHUNK = 128
_NCHUNK = _TPW // _CHUNK
_NBUF = 2


def _gather_body(table_hbm, idx_hbm, out_hbm, idx_v, bufs, gsems, ssems):
    wid = lax.axis_index("s") * 2 + lax.axis_index("c")
    base = wid * _TPW

    pltpu.sync_copy(idx_hbm.at[wid], idx_v)

    def gather_op(chunk, b):
        return pltpu.make_async_copy(
            table_hbm.at[idx_v.at[chunk]], bufs[b], gsems[b])

    def scatter_op(chunk, b):
        off = base + chunk * _CHUNK
        return pltpu.make_async_copy(
            bufs[b], out_hbm.at[pl.ds(off, _CHUNK)], ssems[b])

    for b in range(_NBUF):
        gather_op(b, b).start()

    def group(g, carry):
        for b in range(_NBUF):
            i = g * _NBUF + b
            gather_op(i, b).wait()
            scatter_op(i, b).start()
            scatter_op(i, b).wait()
            gather_op(i + _NBUF, b).start()
        return carry

    lax.fori_loop(0, (_NCHUNK - _NBUF) // _NBUF, group, 0)

    for b in range(_NBUF):
        i = _NCHUNK - _NBUF + b
        gather_op(i, b).wait()
        scatter_op(i, b).start()
    for b in range(_NBUF):
        i = _NCHUNK - _NBUF + b
        scatter_op(i, b).wait()


_gather = functools.partial(
    pl.kernel,
    out_type=jax.ShapeDtypeStruct((TOKENS, EMBED), jnp.float32),
    mesh=plsc.VectorSubcoreMesh(core_axis_name="c", subcore_axis_name="s"),
    scratch_types=[
        pltpu.VMEM((_NCHUNK, _CHUNK), jnp.int32),
        [pltpu.VMEM((_CHUNK, EMBED), jnp.float32) for _ in range(_NBUF)],
        [pltpu.SemaphoreType.DMA for _ in range(_NBUF)],
        [pltpu.SemaphoreType.DMA for _ in range(_NBUF)],
    ],
)(_gather_body)


_TB = 3200


def _proj_body(e_ref, w_ref, b_ref, o_ref):
    acc = jnp.dot(e_ref[...], w_ref[...], preferred_element_type=jnp.float32)
    o_ref[...] = jnp.maximum(acc + b_ref[...], 0.0)


def _project(embs, W, b):
    return pl.pallas_call(
        _proj_body,
        grid=(TOKENS // _TB,),
        in_specs=[
            pl.BlockSpec((_TB, EMBED), lambda i: (i, 0)),
            pl.BlockSpec((EMBED, HIDDEN), lambda i: (0, 0)),
            pl.BlockSpec((1, HIDDEN), lambda i: (0, 0)),
        ],
        out_specs=pl.BlockSpec((_TB, HIDDEN), lambda i: (i, 0)),
        out_shape=jax.ShapeDtypeStruct((TOKENS, HIDDEN), jnp.float32),
    )(embs, W, b.reshape(1, HIDDEN))


def kernel(X, table, W, b):
    idx = X.T.reshape(_NW, _NCHUNK, _CHUNK).astype(jnp.int32)
    embs = _gather(table, idx)
    out = _project(embs, W, b)
    return out.reshape(SEQ, BATCH, HIDDEN).transpose(1, 0, 2)
